# baseline (speedup 1.0000x reference)
import jax
import jax.numpy as jnp
from jax import lax
from jax.experimental import pallas as pl
from jax.experimental.pallas import tpu as pltpu
from jax.experimental.pallas import tpu_sc as plsc

B = 1024
L = 200
E = 64
F = 32
U = 100
NCLS = 1000
UP = 128
G3 = 3 * UP
V = 1000000
PW = 2 * E

VC = 8192

NC = 2
NS = 16
NW = NC * NS
R = B * L
PER_W = R // NW
CHUNK = 128
NCH = PER_W // CHUNK
KF = 5
ROUNDS = NCH // KF


H2 = 1 << 19
NBLK = pl.cdiv(V, VC)


def _tr_body(t0_ref, t1_ref, o_ref):
    o_ref[:, 0:E] = t0_ref[...].T
    o_ref[:, E:PW] = t1_ref[...].T


def _transpose_table(tableT):
    grid = (H2 // VC,)
    return pl.pallas_call(
        _tr_body,
        grid=grid,
        in_specs=[
            pl.BlockSpec((E, VC), lambda i: (0, i)),
            pl.BlockSpec((E, VC),
                         lambda i: (0, jnp.minimum(i + H2 // VC, NBLK - 1))),
        ],
        out_specs=pl.BlockSpec((VC, PW), lambda i: (i, 0)),
        out_shape=jax.ShapeDtypeStruct((H2, PW), jnp.float32),
    )(tableT, tableT)


def _sc_gather_body(table_hbm, idx_hbm, out_hbm, idx_v, bufs, gsem, wsem):
    c = lax.axis_index("c")
    s = lax.axis_index("s")
    wid = s * NC + c
    base = wid * PER_W
    pltpu.sync_copy(idx_hbm.at[wid], idx_v)

    def round_body(r, _):
        ghandles = []
        for j in range(KF):
            ch = r * KF + j
            h = pltpu.async_copy(table_hbm.at[idx_v.at[ch]], bufs.at[j], gsem)
            ghandles.append((h, ch))
        whandles = []
        for j in range(KF):
            h, ch = ghandles[j]
            h.wait()
            wh = pltpu.async_copy(
                bufs.at[j], out_hbm.at[pl.ds(base + ch * CHUNK, CHUNK)], wsem)
            whandles.append(wh)
        for wh in whandles:
            wh.wait()
        return _

    lax.fori_loop(0, ROUNDS, round_body, 0)


_sc_gather_fn = None


def _sc_gather(table, idx):
    global _sc_gather_fn
    if _sc_gather_fn is None:
        _sc_gather_fn = pl.kernel(
            _sc_gather_body,
            mesh=plsc.VectorSubcoreMesh(core_axis_name="c",
                                        subcore_axis_name="s"),
            out_type=jax.ShapeDtypeStruct((R, PW), jnp.float32),
            compiler_params=pltpu.CompilerParams(use_tc_tiling_on_sc=False),
            scratch_types=[
                pltpu.VMEM((NCH, CHUNK), jnp.int32),
                pltpu.VMEM((KF, CHUNK, PW), jnp.float32),
                pltpu.SemaphoreType.DMA,
                pltpu.SemaphoreType.DMA,
            ],
        )
    return _sc_gather_fn(table, idx)


def _tc_body(x_ref, qe_ref, qo_ref, wt_ref, cb_ref, gk_ref, gr_ref,
             bi_ref, br_ref, o_ref, ye_ref, yo_ref, h_ref):
    i = pl.program_id(0)
    f32 = jnp.float32
    bf16 = jnp.bfloat16

    qe = qe_ref[0]
    qo = qo_ref[0]
    xe_raw = x_ref[0, 0]
    xo_raw = x_ref[0, 1]
    xe = jnp.where(qe > 0.5, xe_raw[:, E:PW], xe_raw[:, 0:E]).astype(bf16)
    xo = jnp.where(qo > 0.5, xo_raw[:, E:PW], xo_raw[:, 0:E]).astype(bf16)
    wt = wt_ref[...]
    ye3 = jnp.dot(xe, wt, preferred_element_type=f32)
    yo3 = jnp.dot(xo, wt, preferred_element_type=f32)
    ye3 = jnp.where(i < U, ye3, 0.0)
    yo3 = jnp.where(i < U, yo3, 0.0)

    @pl.when(i == 0)
    def _init():
        ye_ref[1] = jnp.zeros((B, 3 * F), f32)
        yo_ref[1] = jnp.zeros((B, 3 * F), f32)
        h_ref[...] = jnp.zeros((B, UP), f32)

    @pl.when(i > 0)
    def _step():
        ye_m1 = ye_ref[(i + 1) % 2]
        yo_m1 = yo_ref[(i + 1) % 2]
        yo_m2 = yo_ref[i % 2]
        bias = cb_ref[...]
        c_even = jnp.maximum(
            yo_m2[:, 0:F] + ye_m1[:, F:2 * F] + yo_m1[:, 2 * F:3 * F]
            + bias, 0.0)
        c_odd = jnp.maximum(
            ye_m1[:, 0:F] + yo_m1[:, F:2 * F] + ye3[:, 2 * F:3 * F]
            + bias, 0.0)
        xt = jnp.maximum(c_even, c_odd).astype(bf16)
        h = h_ref[...]
        xg = jnp.dot(xt, gk_ref[...], preferred_element_type=f32) + bi_ref[...]
        hg = (jnp.dot(h.astype(bf16), gr_ref[...], preferred_element_type=f32)
              + br_ref[...])
        xz, xr, xh = xg[:, :UP], xg[:, UP:2 * UP], xg[:, 2 * UP:]
        hz, hr, hn = hg[:, :UP], hg[:, UP:2 * UP], hg[:, 2 * UP:]
        z = 1.0 / (1.0 + jnp.exp(-(xz + hz)))
        r = 1.0 / (1.0 + jnp.exp(-(xr + hr)))
        n = jnp.tanh(xh + r * hn)
        h = z * h + (1.0 - z) * n
        h_ref[...] = h

        @pl.when(i == U)
        def _final():
            o_ref[...] = h

    ye_ref[i % 2] = ye3
    yo_ref[i % 2] = yo3


def _dense_body(h_ref, dw_ref, db_ref, o_ref):
    f32 = jnp.float32
    logits = (jnp.dot(h_ref[...].astype(jnp.bfloat16), dw_ref[...],
                      preferred_element_type=f32) + db_ref[...])
    m = jnp.max(logits, axis=-1, keepdims=True)
    e = jnp.exp(logits - m)
    o_ref[...] = e / jnp.sum(e, axis=-1, keepdims=True)


def _pad_gates(w):
    parts = []
    for g in range(3):
        blk = w[..., g * U:(g + 1) * U]
        pad = [(0, 0)] * (w.ndim - 1) + [(0, UP - U)]
        parts.append(jnp.pad(blk, pad))
    return jnp.concatenate(parts, axis=-1)


def kernel(inputs, table, conv_w, conv_b, gru_k, gru_r, gru_b, dense_w, dense_b):
    P = _transpose_table(table.T)

    it = inputs.astype(jnp.int32).T
    idx3 = (it & (H2 - 1)).reshape(NW, NCH, CHUNK)
    q = (it >> 19).astype(jnp.float32).reshape(L, B, 1)
    g = _sc_gather(P, idx3)
    x4 = g.reshape(U, 2, B, PW)

    wt = jnp.concatenate([conv_w[0], conv_w[1], conv_w[2]], axis=1)
    cbias = conv_b.reshape(1, F)
    gk = _pad_gates(gru_k)
    gr = jnp.pad(_pad_gates(gru_r), ((0, UP - U), (0, 0)))
    bi = _pad_gates(gru_b[0]).reshape(1, G3)
    br = _pad_gates(gru_b[1]).reshape(1, G3)
    dw = jnp.pad(dense_w, ((0, UP - U), (0, 0)))
    db = dense_b.reshape(1, NCLS)

    bf16 = jnp.bfloat16
    grid = (U + 1,)
    h_last = pl.pallas_call(
        _tc_body,
        grid=grid,
        in_specs=[
            pl.BlockSpec((1, 2, B, PW),
                         lambda i: (jnp.minimum(i, U - 1), 0, 0, 0)),
            pl.BlockSpec((1, B, 1), lambda i: (2 * jnp.minimum(i, U - 1), 0, 0)),
            pl.BlockSpec((1, B, 1),
                         lambda i: (2 * jnp.minimum(i, U - 1) + 1, 0, 0)),
            pl.BlockSpec((E, 3 * F), lambda i: (0, 0)),
            pl.BlockSpec((1, F), lambda i: (0, 0)),
            pl.BlockSpec((F, G3), lambda i: (0, 0)),
            pl.BlockSpec((UP, G3), lambda i: (0, 0)),
            pl.BlockSpec((1, G3), lambda i: (0, 0)),
            pl.BlockSpec((1, G3), lambda i: (0, 0)),
        ],
        out_specs=pl.BlockSpec((B, UP), lambda i: (0, 0)),
        out_shape=jax.ShapeDtypeStruct((B, UP), jnp.float32),
        scratch_shapes=[pltpu.VMEM((2, B, 3 * F), jnp.float32),
                        pltpu.VMEM((2, B, 3 * F), jnp.float32),
                        pltpu.VMEM((B, UP), jnp.float32)],
    )(x4, q, q, wt.astype(bf16), cbias, gk.astype(bf16), gr.astype(bf16),
      bi, br)
    out = pl.pallas_call(
        _dense_body,
        out_shape=jax.ShapeDtypeStruct((B, NCLS), jnp.float32),
    )(h_last, dw.astype(bf16), db)
    return out


# speedup vs baseline: 1.0473x; 1.0473x over previous
import jax
import jax.numpy as jnp
from jax import lax
from jax.experimental import pallas as pl
from jax.experimental.pallas import tpu as pltpu
from jax.experimental.pallas import tpu_sc as plsc

B = 1024
L = 200
E = 64
F = 32
U = 100
NCLS = 1000
UP = 128
G3 = 3 * UP
V = 1000000
PW = 2 * E

VC = 8192

NC = 2
NS = 16
NW = NC * NS
R = B * L
PER_W = R // NW
CHUNK = 128
NCH = PER_W // CHUNK
KF = 5
ROUNDS = NCH // KF


H2 = 1 << 19
NBLK = pl.cdiv(V, VC)


def _tr_body(t0_ref, t1_ref, o_ref):
    o_ref[:, 0:E] = t0_ref[...].T
    o_ref[:, E:PW] = t1_ref[...].T


def _transpose_table(tableT):
    grid = (H2 // VC,)
    return pl.pallas_call(
        _tr_body,
        grid=grid,
        in_specs=[
            pl.BlockSpec((E, VC), lambda i: (0, i)),
            pl.BlockSpec((E, VC),
                         lambda i: (0, jnp.minimum(i + H2 // VC, NBLK - 1))),
        ],
        out_specs=pl.BlockSpec((VC, PW), lambda i: (i, 0)),
        out_shape=jax.ShapeDtypeStruct((H2, PW), jnp.float32),
    )(tableT, tableT)


def _sc_gather_body(table_hbm, idx_hbm, out_hbm, idx_v, bufs, gsem, wsem):
    c = lax.axis_index("c")
    s = lax.axis_index("s")
    wid = s * NC + c
    base = wid * PER_W
    pltpu.sync_copy(idx_hbm.at[wid], idx_v)

    def round_body(r, _):
        ghandles = []
        for j in range(KF):
            ch = r * KF + j
            h = pltpu.async_copy(table_hbm.at[idx_v.at[ch]], bufs.at[j], gsem)
            ghandles.append((h, ch))
        whandles = []
        for j in range(KF):
            h, ch = ghandles[j]
            h.wait()
            wh = pltpu.async_copy(
                bufs.at[j], out_hbm.at[pl.ds(base + ch * CHUNK, CHUNK)], wsem)
            whandles.append(wh)
        for wh in whandles:
            wh.wait()
        return _

    lax.fori_loop(0, ROUNDS, round_body, 0)


_sc_gather_fn = None


def _sc_gather(table, idx):
    global _sc_gather_fn
    if _sc_gather_fn is None:
        _sc_gather_fn = pl.kernel(
            _sc_gather_body,
            mesh=plsc.VectorSubcoreMesh(core_axis_name="c",
                                        subcore_axis_name="s"),
            out_type=jax.ShapeDtypeStruct((R, PW), jnp.float32),
            compiler_params=pltpu.CompilerParams(use_tc_tiling_on_sc=False),
            scratch_types=[
                pltpu.VMEM((NCH, CHUNK), jnp.int32),
                pltpu.VMEM((KF, CHUNK, PW), jnp.float32),
                pltpu.SemaphoreType.DMA,
                pltpu.SemaphoreType.DMA,
            ],
        )
    return _sc_gather_fn(table, idx)


def _tc_body(x_ref, q_ref, wt_ref, cb_ref, gk_ref, gr_ref,
             bi_ref, br_ref, o_ref, ye_ref, yo_ref, h_ref):
    i = pl.program_id(0)
    f32 = jnp.float32
    bf16 = jnp.bfloat16

    ti = jnp.minimum(i, U - 1)
    tcol = lax.broadcasted_iota(jnp.int32, (L, 2), 0)
    onehot = (tcol == 2 * ti + lax.broadcasted_iota(jnp.int32, (L, 2), 1)
              ).astype(jnp.float32)
    qeo = jnp.dot(q_ref[...], onehot, preferred_element_type=jnp.float32)
    qe = qeo[:, 0:1]
    qo = qeo[:, 1:2]
    xe_raw = x_ref[0, 0]
    xo_raw = x_ref[0, 1]
    xe = jnp.where(qe > 0.5, xe_raw[:, E:PW], xe_raw[:, 0:E]).astype(bf16)
    xo = jnp.where(qo > 0.5, xo_raw[:, E:PW], xo_raw[:, 0:E]).astype(bf16)
    wt = wt_ref[...]
    ye3 = jnp.dot(xe, wt, preferred_element_type=f32)
    yo3 = jnp.dot(xo, wt, preferred_element_type=f32)
    ye3 = jnp.where(i < U, ye3, 0.0)
    yo3 = jnp.where(i < U, yo3, 0.0)

    @pl.when(i == 0)
    def _init():
        ye_ref[1] = jnp.zeros((B, 3 * F), f32)
        yo_ref[1] = jnp.zeros((B, 3 * F), f32)
        h_ref[...] = jnp.zeros((B, UP), f32)

    @pl.when(i > 0)
    def _step():
        ye_m1 = ye_ref[(i + 1) % 2]
        yo_m1 = yo_ref[(i + 1) % 2]
        yo_m2 = yo_ref[i % 2]
        bias = cb_ref[...]
        c_even = jnp.maximum(
            yo_m2[:, 0:F] + ye_m1[:, F:2 * F] + yo_m1[:, 2 * F:3 * F]
            + bias, 0.0)
        c_odd = jnp.maximum(
            ye_m1[:, 0:F] + yo_m1[:, F:2 * F] + ye3[:, 2 * F:3 * F]
            + bias, 0.0)
        xt = jnp.maximum(c_even, c_odd).astype(bf16)
        h = h_ref[...]
        xg = jnp.dot(xt, gk_ref[...], preferred_element_type=f32) + bi_ref[...]
        hg = (jnp.dot(h.astype(bf16), gr_ref[...], preferred_element_type=f32)
              + br_ref[...])
        xz, xr, xh = xg[:, :UP], xg[:, UP:2 * UP], xg[:, 2 * UP:]
        hz, hr, hn = hg[:, :UP], hg[:, UP:2 * UP], hg[:, 2 * UP:]
        z = 1.0 / (1.0 + jnp.exp(-(xz + hz)))
        r = 1.0 / (1.0 + jnp.exp(-(xr + hr)))
        n = jnp.tanh(xh + r * hn)
        h = z * h + (1.0 - z) * n
        h_ref[...] = h

        @pl.when(i == U)
        def _final():
            o_ref[...] = h

    ye_ref[i % 2] = ye3
    yo_ref[i % 2] = yo3


def _dense_body(h_ref, dw_ref, db_ref, o_ref):
    f32 = jnp.float32
    logits = (jnp.dot(h_ref[...].astype(jnp.bfloat16), dw_ref[...],
                      preferred_element_type=f32) + db_ref[...])
    m = jnp.max(logits, axis=-1, keepdims=True)
    e = jnp.exp(logits - m)
    o_ref[...] = e / jnp.sum(e, axis=-1, keepdims=True)


def _pad_gates(w):
    parts = []
    for g in range(3):
        blk = w[..., g * U:(g + 1) * U]
        pad = [(0, 0)] * (w.ndim - 1) + [(0, UP - U)]
        parts.append(jnp.pad(blk, pad))
    return jnp.concatenate(parts, axis=-1)


def kernel(inputs, table, conv_w, conv_b, gru_k, gru_r, gru_b, dense_w, dense_b):
    P = _transpose_table(table.T)

    it = inputs.astype(jnp.int32).T
    idx3 = (it & (H2 - 1)).reshape(NW, NCH, CHUNK)
    q = (inputs.astype(jnp.int32) >> 19).astype(jnp.float32)
    g = _sc_gather(P, idx3)
    x4 = g.reshape(U, 2, B, PW)

    wt = jnp.concatenate([conv_w[0], conv_w[1], conv_w[2]], axis=1)
    cbias = conv_b.reshape(1, F)
    gk = _pad_gates(gru_k)
    gr = jnp.pad(_pad_gates(gru_r), ((0, UP - U), (0, 0)))
    bi = _pad_gates(gru_b[0]).reshape(1, G3)
    br = _pad_gates(gru_b[1]).reshape(1, G3)
    dw = jnp.pad(dense_w, ((0, UP - U), (0, 0)))
    db = dense_b.reshape(1, NCLS)

    bf16 = jnp.bfloat16
    grid = (U + 1,)
    h_last = pl.pallas_call(
        _tc_body,
        grid=grid,
        in_specs=[
            pl.BlockSpec((1, 2, B, PW),
                         lambda i: (jnp.minimum(i, U - 1), 0, 0, 0)),
            pl.BlockSpec((B, L), lambda i: (0, 0)),
            pl.BlockSpec((E, 3 * F), lambda i: (0, 0)),
            pl.BlockSpec((1, F), lambda i: (0, 0)),
            pl.BlockSpec((F, G3), lambda i: (0, 0)),
            pl.BlockSpec((UP, G3), lambda i: (0, 0)),
            pl.BlockSpec((1, G3), lambda i: (0, 0)),
            pl.BlockSpec((1, G3), lambda i: (0, 0)),
        ],
        out_specs=pl.BlockSpec((B, UP), lambda i: (0, 0)),
        out_shape=jax.ShapeDtypeStruct((B, UP), jnp.float32),
        scratch_shapes=[pltpu.VMEM((2, B, 3 * F), jnp.float32),
                        pltpu.VMEM((2, B, 3 * F), jnp.float32),
                        pltpu.VMEM((B, UP), jnp.float32)],
    )(x4, q, wt.astype(bf16), cbias, gk.astype(bf16), gr.astype(bf16),
      bi, br)
    out = pl.pallas_call(
        _dense_body,
        out_shape=jax.ShapeDtypeStruct((B, NCLS), jnp.float32),
    )(h_last, dw.astype(bf16), db)
    return out


# speedup vs baseline: 1.1152x; 1.0648x over previous
import jax
import jax.numpy as jnp
from jax import lax
from jax.experimental import pallas as pl
from jax.experimental.pallas import tpu as pltpu
from jax.experimental.pallas import tpu_sc as plsc

B = 1024
L = 200
E = 64
F = 32
U = 100
NCLS = 1000
UP = 128
G3 = 3 * UP
V = 1000000
PW = 2 * E

VC = 8192

NC = 2
NS = 16
NW = NC * NS
R = B * L
PER_W = R // NW
CHUNK = 128
NCH = PER_W // CHUNK
KF = 5
ROUNDS = NCH // KF


def _tr_body(t_ref, o_ref):
    t = t_ref[...]
    o_ref[:, 0:E] = t.T
    o_ref[:, E:PW] = jnp.zeros((VC, E), jnp.float32)


def _transpose_table(tableT):
    grid = (pl.cdiv(V, VC),)
    return pl.pallas_call(
        _tr_body,
        grid=grid,
        in_specs=[pl.BlockSpec((E, VC), lambda i: (0, i))],
        out_specs=pl.BlockSpec((VC, PW), lambda i: (i, 0)),
        out_shape=jax.ShapeDtypeStruct((V, PW), jnp.float32),
    )(tableT)


def _sc_gather_body(table_hbm, idx_hbm, out_hbm, idx_v, bufs, gsem, wsem):
    c = lax.axis_index("c")
    s = lax.axis_index("s")
    wid = s * NC + c
    base = wid * PER_W
    pltpu.sync_copy(idx_hbm.at[wid], idx_v)

    def round_body(r, _):
        ghandles = []
        for j in range(KF):
            ch = r * KF + j
            h = pltpu.async_copy(table_hbm.at[idx_v.at[ch]], bufs.at[j], gsem)
            ghandles.append((h, ch))
        whandles = []
        for j in range(KF):
            h, ch = ghandles[j]
            h.wait()
            wh = pltpu.async_copy(
                bufs.at[j], out_hbm.at[pl.ds(base + ch * CHUNK, CHUNK)], wsem)
            whandles.append(wh)
        for wh in whandles:
            wh.wait()
        return _

    lax.fori_loop(0, ROUNDS, round_body, 0)


_sc_gather_fn = None


def _sc_gather(table, idx):
    global _sc_gather_fn
    if _sc_gather_fn is None:
        _sc_gather_fn = pl.kernel(
            _sc_gather_body,
            mesh=plsc.VectorSubcoreMesh(core_axis_name="c",
                                        subcore_axis_name="s"),
            out_type=jax.ShapeDtypeStruct((R, PW), jnp.float32),
            compiler_params=pltpu.CompilerParams(use_tc_tiling_on_sc=False),
            scratch_types=[
                pltpu.VMEM((NCH, CHUNK), jnp.int32),
                pltpu.VMEM((KF, CHUNK, PW), jnp.float32),
                pltpu.SemaphoreType.DMA,
                pltpu.SemaphoreType.DMA,
            ],
        )
    return _sc_gather_fn(table, idx)


def _tc_body(x_ref, wt_ref, cb_ref, gk_ref, gr_ref,
             bi_ref, br_ref, o_ref, ye_ref, yo_ref, h_ref):
    i = pl.program_id(0)
    f32 = jnp.float32
    bf16 = jnp.bfloat16

    xe = x_ref[0, 0].astype(bf16)
    xo = x_ref[0, 1].astype(bf16)
    wt = wt_ref[...]
    ye3 = jnp.dot(xe, wt, preferred_element_type=f32)
    yo3 = jnp.dot(xo, wt, preferred_element_type=f32)
    ye3 = jnp.where(i < U, ye3, 0.0)
    yo3 = jnp.where(i < U, yo3, 0.0)

    @pl.when(i == 0)
    def _init():
        ye_ref[1] = jnp.zeros((B, 3 * F), f32)
        yo_ref[1] = jnp.zeros((B, 3 * F), f32)
        h_ref[...] = jnp.zeros((B, UP), f32)

    @pl.when(i > 0)
    def _step():
        ye_m1 = ye_ref[(i + 1) % 2]
        yo_m1 = yo_ref[(i + 1) % 2]
        yo_m2 = yo_ref[i % 2]
        bias = cb_ref[...]
        c_even = jnp.maximum(
            yo_m2[:, 0:F] + ye_m1[:, F:2 * F] + yo_m1[:, 2 * F:3 * F]
            + bias, 0.0)
        c_odd = jnp.maximum(
            ye_m1[:, 0:F] + yo_m1[:, F:2 * F] + ye3[:, 2 * F:3 * F]
            + bias, 0.0)
        xt = jnp.maximum(c_even, c_odd).astype(bf16)
        h = h_ref[...]
        xg = jnp.dot(xt, gk_ref[...], preferred_element_type=f32) + bi_ref[...]
        hg = (jnp.dot(h.astype(bf16), gr_ref[...], preferred_element_type=f32)
              + br_ref[...])
        xz, xr, xh = xg[:, :UP], xg[:, UP:2 * UP], xg[:, 2 * UP:]
        hz, hr, hn = hg[:, :UP], hg[:, UP:2 * UP], hg[:, 2 * UP:]
        z = 1.0 / (1.0 + jnp.exp(-(xz + hz)))
        r = 1.0 / (1.0 + jnp.exp(-(xr + hr)))
        n = jnp.tanh(xh + r * hn)
        h = z * h + (1.0 - z) * n
        h_ref[...] = h

        @pl.when(i == U)
        def _final():
            o_ref[...] = h

    ye_ref[i % 2] = ye3
    yo_ref[i % 2] = yo3


def _dense_body(h_ref, dw_ref, db_ref, o_ref):
    f32 = jnp.float32
    logits = (jnp.dot(h_ref[...].astype(jnp.bfloat16), dw_ref[...],
                      preferred_element_type=f32) + db_ref[...])
    m = jnp.max(logits, axis=-1, keepdims=True)
    e = jnp.exp(logits - m)
    o_ref[...] = e / jnp.sum(e, axis=-1, keepdims=True)


def _pad_gates(w):
    parts = []
    for g in range(3):
        blk = w[..., g * U:(g + 1) * U]
        pad = [(0, 0)] * (w.ndim - 1) + [(0, UP - U)]
        parts.append(jnp.pad(blk, pad))
    return jnp.concatenate(parts, axis=-1)


def kernel(inputs, table, conv_w, conv_b, gru_k, gru_r, gru_b, dense_w, dense_b):
    P = _transpose_table(table.T)

    idx3 = inputs.astype(jnp.int32).T.reshape(NW, NCH, CHUNK)
    g = _sc_gather(P, idx3)
    x4 = g.reshape(U, 2, B, PW)

    wt = jnp.zeros((PW, 3 * F), jnp.float32)
    for k in range(3):
        wt = wt.at[0:E, F * k:F * (k + 1)].set(conv_w[k])
    cbias = conv_b.reshape(1, F)
    gk = _pad_gates(gru_k)
    gr = jnp.pad(_pad_gates(gru_r), ((0, UP - U), (0, 0)))
    bi = _pad_gates(gru_b[0]).reshape(1, G3)
    br = _pad_gates(gru_b[1]).reshape(1, G3)
    dw = jnp.pad(dense_w, ((0, UP - U), (0, 0)))
    db = dense_b.reshape(1, NCLS)

    bf16 = jnp.bfloat16
    grid = (U + 1,)
    h_last = pl.pallas_call(
        _tc_body,
        grid=grid,
        in_specs=[
            pl.BlockSpec((1, 2, B, PW),
                         lambda i: (jnp.minimum(i, U - 1), 0, 0, 0)),
            pl.BlockSpec((PW, 3 * F), lambda i: (0, 0)),
            pl.BlockSpec((1, F), lambda i: (0, 0)),
            pl.BlockSpec((F, G3), lambda i: (0, 0)),
            pl.BlockSpec((UP, G3), lambda i: (0, 0)),
            pl.BlockSpec((1, G3), lambda i: (0, 0)),
            pl.BlockSpec((1, G3), lambda i: (0, 0)),
        ],
        out_specs=pl.BlockSpec((B, UP), lambda i: (0, 0)),
        out_shape=jax.ShapeDtypeStruct((B, UP), jnp.float32),
        scratch_shapes=[pltpu.VMEM((2, B, 3 * F), jnp.float32),
                        pltpu.VMEM((2, B, 3 * F), jnp.float32),
                        pltpu.VMEM((B, UP), jnp.float32)],
    )(x4, wt.astype(bf16), cbias, gk.astype(bf16), gr.astype(bf16),
      bi, br)
    out = pl.pallas_call(
        _dense_body,
        out_shape=jax.ShapeDtypeStruct((B, NCLS), jnp.float32),
    )(h_last, dw.astype(bf16), db)
    return out


# speedup vs baseline: 1.1589x; 1.0392x over previous
import jax
import jax.numpy as jnp
from jax import lax
from jax.experimental import pallas as pl
from jax.experimental.pallas import tpu as pltpu
from jax.experimental.pallas import tpu_sc as plsc

B = 1024
L = 200
E = 64
F = 32
U = 100
NCLS = 1000
UP = 128
G3 = 3 * UP
V = 1000000
PW = 2 * E

VC = 16384

NC = 2
NS = 16
NW = NC * NS
R = B * L
PER_W = R // NW
CHUNK = 128
NCH = PER_W // CHUNK
KF = 5
ROUNDS = NCH // KF


def _tr_body(t_ref, o_ref):
    t = t_ref[...]
    o_ref[:, 0:E] = t.T
    o_ref[:, E:PW] = jnp.zeros((VC, E), jnp.float32)


def _transpose_table(tableT):
    grid = (pl.cdiv(V, VC),)
    return pl.pallas_call(
        _tr_body,
        grid=grid,
        in_specs=[pl.BlockSpec((E, VC), lambda i: (0, i))],
        out_specs=pl.BlockSpec((VC, PW), lambda i: (i, 0)),
        out_shape=jax.ShapeDtypeStruct((V, PW), jnp.float32),
    )(tableT)


def _sc_gather_body(table_hbm, idx_hbm, out_hbm, idx_v, bufs, gsem, wsem):
    c = lax.axis_index("c")
    s = lax.axis_index("s")
    wid = s * NC + c
    base = wid * PER_W
    pltpu.sync_copy(idx_hbm.at[wid], idx_v)

    def round_body(r, _):
        ghandles = []
        for j in range(KF):
            ch = r * KF + j
            h = pltpu.async_copy(table_hbm.at[idx_v.at[ch]], bufs.at[j], gsem)
            ghandles.append((h, ch))
        whandles = []
        for j in range(KF):
            h, ch = ghandles[j]
            h.wait()
            wh = pltpu.async_copy(
                bufs.at[j], out_hbm.at[pl.ds(base + ch * CHUNK, CHUNK)], wsem)
            whandles.append(wh)
        for wh in whandles:
            wh.wait()
        return _

    lax.fori_loop(0, ROUNDS, round_body, 0)


_sc_gather_fn = None


def _sc_gather(table, idx):
    global _sc_gather_fn
    if _sc_gather_fn is None:
        _sc_gather_fn = pl.kernel(
            _sc_gather_body,
            mesh=plsc.VectorSubcoreMesh(core_axis_name="c",
                                        subcore_axis_name="s"),
            out_type=jax.ShapeDtypeStruct((R, PW), jnp.float32),
            compiler_params=pltpu.CompilerParams(use_tc_tiling_on_sc=False),
            scratch_types=[
                pltpu.VMEM((NCH, CHUNK), jnp.int32),
                pltpu.VMEM((KF, CHUNK, PW), jnp.float32),
                pltpu.SemaphoreType.DMA,
                pltpu.SemaphoreType.DMA,
            ],
        )
    return _sc_gather_fn(table, idx)


def _tc_body(x_ref, wt_ref, cb_ref, gk_ref, gr_ref,
             bi_ref, br_ref, o_ref, ye_ref, yo_ref, h_ref):
    i = pl.program_id(0)
    f32 = jnp.float32
    bf16 = jnp.bfloat16

    xe = x_ref[0, 0].astype(bf16)
    xo = x_ref[0, 1].astype(bf16)
    wt = wt_ref[...]
    ye3 = jnp.dot(xe, wt, preferred_element_type=f32)
    yo3 = jnp.dot(xo, wt, preferred_element_type=f32)
    ye3 = jnp.where(i < U, ye3, 0.0)
    yo3 = jnp.where(i < U, yo3, 0.0)

    @pl.when(i == 0)
    def _init():
        ye_ref[1] = jnp.zeros((B, 3 * F), f32)
        yo_ref[1] = jnp.zeros((B, 3 * F), f32)
        h_ref[...] = jnp.zeros((B, UP), f32)

    @pl.when(i > 0)
    def _step():
        ye_m1 = ye_ref[(i + 1) % 2]
        yo_m1 = yo_ref[(i + 1) % 2]
        yo_m2 = yo_ref[i % 2]
        bias = cb_ref[...]
        c_even = jnp.maximum(
            yo_m2[:, 0:F] + ye_m1[:, F:2 * F] + yo_m1[:, 2 * F:3 * F]
            + bias, 0.0)
        c_odd = jnp.maximum(
            ye_m1[:, 0:F] + yo_m1[:, F:2 * F] + ye3[:, 2 * F:3 * F]
            + bias, 0.0)
        xt = jnp.maximum(c_even, c_odd).astype(bf16)
        h = h_ref[...]
        xg = jnp.dot(xt, gk_ref[...], preferred_element_type=f32) + bi_ref[...]
        hg = (jnp.dot(h.astype(bf16), gr_ref[...], preferred_element_type=f32)
              + br_ref[...])
        xz, xr, xh = xg[:, :UP], xg[:, UP:2 * UP], xg[:, 2 * UP:]
        hz, hr, hn = hg[:, :UP], hg[:, UP:2 * UP], hg[:, 2 * UP:]
        z = 1.0 / (1.0 + jnp.exp(-(xz + hz)))
        r = 1.0 / (1.0 + jnp.exp(-(xr + hr)))
        n = jnp.tanh(xh + r * hn)
        h = z * h + (1.0 - z) * n
        h_ref[...] = h

        @pl.when(i == U)
        def _final():
            o_ref[...] = h

    ye_ref[i % 2] = ye3
    yo_ref[i % 2] = yo3


def _dense_body(h_ref, dw_ref, db_ref, o_ref):
    f32 = jnp.float32
    logits = (jnp.dot(h_ref[...].astype(jnp.bfloat16), dw_ref[...],
                      preferred_element_type=f32) + db_ref[...])
    m = jnp.max(logits, axis=-1, keepdims=True)
    e = jnp.exp(logits - m)
    o_ref[...] = e / jnp.sum(e, axis=-1, keepdims=True)


def _pad_gates(w):
    parts = []
    for g in range(3):
        blk = w[..., g * U:(g + 1) * U]
        pad = [(0, 0)] * (w.ndim - 1) + [(0, UP - U)]
        parts.append(jnp.pad(blk, pad))
    return jnp.concatenate(parts, axis=-1)


def kernel(inputs, table, conv_w, conv_b, gru_k, gru_r, gru_b, dense_w, dense_b):
    P = _transpose_table(table.T)

    idx3 = inputs.astype(jnp.int32).T.reshape(NW, NCH, CHUNK)
    g = _sc_gather(P, idx3)
    x4 = g.reshape(U, 2, B, PW)

    wt = jnp.zeros((PW, 3 * F), jnp.float32)
    for k in range(3):
        wt = wt.at[0:E, F * k:F * (k + 1)].set(conv_w[k])
    cbias = conv_b.reshape(1, F)
    gk = _pad_gates(gru_k)
    gr = jnp.pad(_pad_gates(gru_r), ((0, UP - U), (0, 0)))
    bi = _pad_gates(gru_b[0]).reshape(1, G3)
    br = _pad_gates(gru_b[1]).reshape(1, G3)
    dw = jnp.pad(dense_w, ((0, UP - U), (0, 0)))
    db = dense_b.reshape(1, NCLS)

    bf16 = jnp.bfloat16
    grid = (U + 1,)
    h_last = pl.pallas_call(
        _tc_body,
        grid=grid,
        in_specs=[
            pl.BlockSpec((1, 2, B, PW),
                         lambda i: (jnp.minimum(i, U - 1), 0, 0, 0)),
            pl.BlockSpec((PW, 3 * F), lambda i: (0, 0)),
            pl.BlockSpec((1, F), lambda i: (0, 0)),
            pl.BlockSpec((F, G3), lambda i: (0, 0)),
            pl.BlockSpec((UP, G3), lambda i: (0, 0)),
            pl.BlockSpec((1, G3), lambda i: (0, 0)),
            pl.BlockSpec((1, G3), lambda i: (0, 0)),
        ],
        out_specs=pl.BlockSpec((B, UP), lambda i: (0, 0)),
        out_shape=jax.ShapeDtypeStruct((B, UP), jnp.float32),
        scratch_shapes=[pltpu.VMEM((2, B, 3 * F), jnp.float32),
                        pltpu.VMEM((2, B, 3 * F), jnp.float32),
                        pltpu.VMEM((B, UP), jnp.float32)],
    )(x4, wt.astype(bf16), cbias, gk.astype(bf16), gr.astype(bf16),
      bi, br)
    out = pl.pallas_call(
        _dense_body,
        out_shape=jax.ShapeDtypeStruct((B, NCLS), jnp.float32),
    )(h_last, dw.astype(bf16), db)
    return out


# speedup vs baseline: 1.1689x; 1.0086x over previous
import jax
import jax.numpy as jnp
from jax import lax
from jax.experimental import pallas as pl
from jax.experimental.pallas import tpu as pltpu
from jax.experimental.pallas import tpu_sc as plsc

B = 1024
L = 200
E = 64
F = 32
U = 100
NCLS = 1000
UP = 128
G3 = 3 * UP
V = 1000000
PW = 2 * E

VC = 32768

NC = 2
NS = 16
NW = NC * NS
R = B * L
PER_W = R // NW
CHUNK = 128
NCH = PER_W // CHUNK
KF = 5
ROUNDS = NCH // KF


def _tr_body(t_ref, o_ref):
    t = t_ref[...]
    o_ref[:, 0:E] = t.T
    o_ref[:, E:PW] = jnp.zeros((VC, E), jnp.float32)


def _transpose_table(tableT):
    grid = (pl.cdiv(V, VC),)
    return pl.pallas_call(
        _tr_body,
        grid=grid,
        in_specs=[pl.BlockSpec((E, VC), lambda i: (0, i))],
        out_specs=pl.BlockSpec((VC, PW), lambda i: (i, 0)),
        out_shape=jax.ShapeDtypeStruct((V, PW), jnp.float32),
    )(tableT)


def _sc_gather_body(table_hbm, idx_hbm, out_hbm, idx_v, bufs, gsem, wsem):
    c = lax.axis_index("c")
    s = lax.axis_index("s")
    wid = s * NC + c
    base = wid * PER_W
    pltpu.sync_copy(idx_hbm.at[wid], idx_v)

    def round_body(r, _):
        ghandles = []
        for j in range(KF):
            ch = r * KF + j
            h = pltpu.async_copy(table_hbm.at[idx_v.at[ch]], bufs.at[j], gsem)
            ghandles.append((h, ch))
        whandles = []
        for j in range(KF):
            h, ch = ghandles[j]
            h.wait()
            wh = pltpu.async_copy(
                bufs.at[j], out_hbm.at[pl.ds(base + ch * CHUNK, CHUNK)], wsem)
            whandles.append(wh)
        for wh in whandles:
            wh.wait()
        return _

    lax.fori_loop(0, ROUNDS, round_body, 0)


_sc_gather_fn = None


def _sc_gather(table, idx):
    global _sc_gather_fn
    if _sc_gather_fn is None:
        _sc_gather_fn = pl.kernel(
            _sc_gather_body,
            mesh=plsc.VectorSubcoreMesh(core_axis_name="c",
                                        subcore_axis_name="s"),
            out_type=jax.ShapeDtypeStruct((R, PW), jnp.float32),
            compiler_params=pltpu.CompilerParams(use_tc_tiling_on_sc=False),
            scratch_types=[
                pltpu.VMEM((NCH, CHUNK), jnp.int32),
                pltpu.VMEM((KF, CHUNK, PW), jnp.float32),
                pltpu.SemaphoreType.DMA,
                pltpu.SemaphoreType.DMA,
            ],
        )
    return _sc_gather_fn(table, idx)


def _tc_body(x_ref, wt_ref, cb_ref, gk_ref, gr_ref,
             bi_ref, br_ref, o_ref, ye_ref, yo_ref, h_ref):
    i = pl.program_id(0)
    f32 = jnp.float32
    bf16 = jnp.bfloat16

    xe = x_ref[0, 0].astype(bf16)
    xo = x_ref[0, 1].astype(bf16)
    wt = wt_ref[...]
    ye3 = jnp.dot(xe, wt, preferred_element_type=f32)
    yo3 = jnp.dot(xo, wt, preferred_element_type=f32)
    ye3 = jnp.where(i < U, ye3, 0.0)
    yo3 = jnp.where(i < U, yo3, 0.0)

    @pl.when(i == 0)
    def _init():
        ye_ref[1] = jnp.zeros((B, 3 * F), f32)
        yo_ref[1] = jnp.zeros((B, 3 * F), f32)
        h_ref[...] = jnp.zeros((B, UP), f32)

    @pl.when(i > 0)
    def _step():
        ye_m1 = ye_ref[(i + 1) % 2]
        yo_m1 = yo_ref[(i + 1) % 2]
        yo_m2 = yo_ref[i % 2]
        bias = cb_ref[...]
        c_even = jnp.maximum(
            yo_m2[:, 0:F] + ye_m1[:, F:2 * F] + yo_m1[:, 2 * F:3 * F]
            + bias, 0.0)
        c_odd = jnp.maximum(
            ye_m1[:, 0:F] + yo_m1[:, F:2 * F] + ye3[:, 2 * F:3 * F]
            + bias, 0.0)
        xt = jnp.maximum(c_even, c_odd).astype(bf16)
        h = h_ref[...]
        xg = jnp.dot(xt, gk_ref[...], preferred_element_type=f32) + bi_ref[...]
        hg = (jnp.dot(h.astype(bf16), gr_ref[...], preferred_element_type=f32)
              + br_ref[...])
        xz, xr, xh = xg[:, :UP], xg[:, UP:2 * UP], xg[:, 2 * UP:]
        hz, hr, hn = hg[:, :UP], hg[:, UP:2 * UP], hg[:, 2 * UP:]
        z = 1.0 / (1.0 + jnp.exp(-(xz + hz)))
        r = 1.0 / (1.0 + jnp.exp(-(xr + hr)))
        n = jnp.tanh(xh + r * hn)
        h = z * h + (1.0 - z) * n
        h_ref[...] = h

        @pl.when(i == U)
        def _final():
            o_ref[...] = h

    ye_ref[i % 2] = ye3
    yo_ref[i % 2] = yo3


def _dense_body(h_ref, dw_ref, db_ref, o_ref):
    f32 = jnp.float32
    logits = (jnp.dot(h_ref[...].astype(jnp.bfloat16), dw_ref[...],
                      preferred_element_type=f32) + db_ref[...])
    m = jnp.max(logits, axis=-1, keepdims=True)
    e = jnp.exp(logits - m)
    o_ref[...] = e / jnp.sum(e, axis=-1, keepdims=True)


def _pad_gates(w):
    parts = []
    for g in range(3):
        blk = w[..., g * U:(g + 1) * U]
        pad = [(0, 0)] * (w.ndim - 1) + [(0, UP - U)]
        parts.append(jnp.pad(blk, pad))
    return jnp.concatenate(parts, axis=-1)


def kernel(inputs, table, conv_w, conv_b, gru_k, gru_r, gru_b, dense_w, dense_b):
    P = _transpose_table(table.T)

    idx3 = inputs.astype(jnp.int32).T.reshape(NW, NCH, CHUNK)
    g = _sc_gather(P, idx3)
    x4 = g.reshape(U, 2, B, PW)

    wt = jnp.zeros((PW, 3 * F), jnp.float32)
    for k in range(3):
        wt = wt.at[0:E, F * k:F * (k + 1)].set(conv_w[k])
    cbias = conv_b.reshape(1, F)
    gk = _pad_gates(gru_k)
    gr = jnp.pad(_pad_gates(gru_r), ((0, UP - U), (0, 0)))
    bi = _pad_gates(gru_b[0]).reshape(1, G3)
    br = _pad_gates(gru_b[1]).reshape(1, G3)
    dw = jnp.pad(dense_w, ((0, UP - U), (0, 0)))
    db = dense_b.reshape(1, NCLS)

    bf16 = jnp.bfloat16
    grid = (U + 1,)
    h_last = pl.pallas_call(
        _tc_body,
        grid=grid,
        in_specs=[
            pl.BlockSpec((1, 2, B, PW),
                         lambda i: (jnp.minimum(i, U - 1), 0, 0, 0)),
            pl.BlockSpec((PW, 3 * F), lambda i: (0, 0)),
            pl.BlockSpec((1, F), lambda i: (0, 0)),
            pl.BlockSpec((F, G3), lambda i: (0, 0)),
            pl.BlockSpec((UP, G3), lambda i: (0, 0)),
            pl.BlockSpec((1, G3), lambda i: (0, 0)),
            pl.BlockSpec((1, G3), lambda i: (0, 0)),
        ],
        out_specs=pl.BlockSpec((B, UP), lambda i: (0, 0)),
        out_shape=jax.ShapeDtypeStruct((B, UP), jnp.float32),
        scratch_shapes=[pltpu.VMEM((2, B, 3 * F), jnp.float32),
                        pltpu.VMEM((2, B, 3 * F), jnp.float32),
                        pltpu.VMEM((B, UP), jnp.float32)],
    )(x4, wt.astype(bf16), cbias, gk.astype(bf16), gr.astype(bf16),
      bi, br)
    out = pl.pallas_call(
        _dense_body,
        out_shape=jax.ShapeDtypeStruct((B, NCLS), jnp.float32),
    )(h_last, dw.astype(bf16), db)
    return out
